# baseline (speedup 1.0000x reference)
import jax
import jax.numpy as jnp
from jax import lax
from jax.experimental import pallas as pl
from jax.experimental.pallas import tpu as pltpu
from jax.experimental.pallas import tpu_sc as plsc

_B = 128
_TAB = 1000
_LANES = 16


def _sc_gather_body(t_hbm, sac_hbm, som_hbm, a_hbm, s_hbm,
                    t_v, sac_v, som_v, a_v, s_v):
    cid = lax.axis_index("c")
    sid = lax.axis_index("s")

    @pl.when(jnp.logical_and(cid == 0, sid == 0))
    def _():
        pltpu.sync_copy(t_hbm, t_v)
        pltpu.sync_copy(sac_hbm, sac_v)
        pltpu.sync_copy(som_hbm, som_v)
        for i in range(_B // _LANES):
            idx = t_v[pl.ds(i * _LANES, _LANES)]
            a_v[pl.ds(i * _LANES, _LANES)] = plsc.load_gather(sac_v, [idx])
            s_v[pl.ds(i * _LANES, _LANES)] = plsc.load_gather(som_v, [idx])
        pltpu.sync_copy(a_v, a_hbm)
        pltpu.sync_copy(s_v, s_hbm)


_SC_GATHER_CACHE = []


def _sc_gather():
    if not _SC_GATHER_CACHE:
        _SC_GATHER_CACHE.append(pl.kernel(
            _sc_gather_body,
            out_type=(jax.ShapeDtypeStruct((_B,), jnp.float32),
                      jax.ShapeDtypeStruct((_B,), jnp.float32)),
            mesh=plsc.VectorSubcoreMesh(core_axis_name="c",
                                        subcore_axis_name="s"),
            compiler_params=pltpu.CompilerParams(needs_layout_passes=False),
            scratch_types=[
                pltpu.VMEM((_B,), jnp.int32),
                pltpu.VMEM((_TAB,), jnp.float32),
                pltpu.VMEM((_TAB,), jnp.float32),
                pltpu.VMEM((_B,), jnp.float32),
                pltpu.VMEM((_B,), jnp.float32),
            ],
        ))
    return _SC_GATHER_CACHE[0]


_ROWS = 3 * 64


def _tc_fma_body(a_sref, s_sref, x_ref, n_ref, o_ref):
    i = pl.program_id(0)
    o_ref[...] = a_sref[i] * x_ref[...] + s_sref[i] * n_ref[...]


def _tc_fma(a, s, x2, n2):
    grid_spec = pltpu.PrefetchScalarGridSpec(
        num_scalar_prefetch=2,
        grid=(_B,),
        in_specs=[
            pl.BlockSpec((_ROWS, 64), lambda i, a_s, s_s: (i, 0)),
            pl.BlockSpec((_ROWS, 64), lambda i, a_s, s_s: (i, 0)),
        ],
        out_specs=pl.BlockSpec((_ROWS, 64), lambda i, a_s, s_s: (i, 0)),
    )
    return pl.pallas_call(
        _tc_fma_body,
        grid_spec=grid_spec,
        out_shape=jax.ShapeDtypeStruct((_B * _ROWS, 64), jnp.float32),
    )(a, s, x2, n2)


def kernel(x0, t, noise, sqrt_alphas_cumprod, sqrt_one_minus_alphas_cumprod):
    a, s = _sc_gather()(t.astype(jnp.int32), sqrt_alphas_cumprod,
                        sqrt_one_minus_alphas_cumprod)
    x2 = x0.reshape(_B * _ROWS, 64)
    n2 = noise.reshape(_B * _ROWS, 64)
    x_t = _tc_fma(a, s, x2, n2).reshape(x0.shape)
    return (x_t, noise)
